# Initial kernel scaffold; baseline (speedup 1.0000x reference)
import jax
import jax.numpy as jnp
from jax.experimental import pallas as pl


def kernel(token_ids, weight):
    raise NotImplementedError("write your pallas kernel here")


# speedup vs baseline: 1.8311x; 1.8311x over previous
import functools

import jax
import jax.numpy as jnp
from jax import lax
from jax.experimental import pallas as pl
from jax.experimental.pallas import tpu as pltpu
from jax.experimental.pallas import tpu_sc as plsc

_NUM_CORES = 2
_NUM_SUBCORES = 16
_NW = _NUM_CORES * _NUM_SUBCORES

_B = 16384 * 50
_D = 64
_B_PER_W = _B // _NW
_CHUNK = 512
_N_CHUNKS = _B_PER_W // _CHUNK

_mesh = plsc.VectorSubcoreMesh(
    core_axis_name="c",
    subcore_axis_name="s",
    num_cores=_NUM_CORES,
    num_subcores=_NUM_SUBCORES,
)


@functools.partial(
    pl.kernel,
    out_type=jax.ShapeDtypeStruct((_B, _D), jnp.float32),
    mesh=_mesh,
    compiler_params=pltpu.CompilerParams(use_tc_tiling_on_sc=False),
    scratch_types=[
        pltpu.VMEM((_B_PER_W,), jnp.int32),
        pltpu.VMEM((_CHUNK, _D), jnp.float32),
        pltpu.SemaphoreType.DMA,
    ],
)
def _gather_kernel(table_hbm, idx_hbm, out_hbm, idx_v, rows_v, sem):
    wid = lax.axis_index("s") * _NUM_CORES + lax.axis_index("c")
    base = wid * _B_PER_W
    pltpu.sync_copy(idx_hbm.at[pl.ds(base, _B_PER_W)], idx_v)

    def chunk_body(i, carry):
        off = i * _CHUNK
        pltpu.async_copy(
            table_hbm.at[idx_v.at[pl.ds(off, _CHUNK)]], rows_v, sem
        ).wait()
        pltpu.sync_copy(rows_v, out_hbm.at[pl.ds(base + off, _CHUNK)])
        return carry

    lax.fori_loop(0, _N_CHUNKS, chunk_body, 0)


def kernel(token_ids, weight):
    flat = token_ids.reshape(-1)
    out = _gather_kernel(weight, flat)
    return out.reshape(token_ids.shape + (weight.shape[1],))


# speedup vs baseline: 1.8758x; 1.0244x over previous
import functools

import jax
import jax.numpy as jnp
from jax import lax
from jax.experimental import pallas as pl
from jax.experimental.pallas import tpu as pltpu
from jax.experimental.pallas import tpu_sc as plsc

_NUM_CORES = 2
_NUM_SUBCORES = 16
_NW = _NUM_CORES * _NUM_SUBCORES

_B = 16384 * 50
_D = 64
_B_PER_W = _B // _NW
_CHUNK = 512
_N_CHUNKS = _B_PER_W // _CHUNK

_mesh = plsc.VectorSubcoreMesh(
    core_axis_name="c",
    subcore_axis_name="s",
    num_cores=_NUM_CORES,
    num_subcores=_NUM_SUBCORES,
)


@functools.partial(
    pl.kernel,
    out_type=jax.ShapeDtypeStruct((_B, _D), jnp.float32),
    mesh=_mesh,
    compiler_params=pltpu.CompilerParams(use_tc_tiling_on_sc=False),
    scratch_types=[
        pltpu.VMEM((_B_PER_W,), jnp.int32),
        pltpu.VMEM((_CHUNK, _D), jnp.float32),
        pltpu.VMEM((_CHUNK, _D), jnp.float32),
        pltpu.SemaphoreType.DMA,
        pltpu.SemaphoreType.DMA,
        pltpu.SemaphoreType.DMA,
        pltpu.SemaphoreType.DMA,
    ],
)
def _gather_kernel(table_hbm, idx_hbm, out_hbm, idx_v, rows0, rows1,
                   gsem0, gsem1, wsem0, wsem1):
    wid = lax.axis_index("s") * _NUM_CORES + lax.axis_index("c")
    base = wid * _B_PER_W
    pltpu.sync_copy(idx_hbm.at[pl.ds(base, _B_PER_W)], idx_v)

    bufs = (rows0, rows1)
    gsems = (gsem0, gsem1)
    wsems = (wsem0, wsem1)

    def start_gather(i, slot):
        pltpu.async_copy(
            table_hbm.at[idx_v.at[pl.ds(i * _CHUNK, _CHUNK)]],
            bufs[slot], gsems[slot])

    def wait_gather(slot):
        pltpu.make_async_copy(
            table_hbm.at[idx_v.at[pl.ds(0, _CHUNK)]],
            bufs[slot], gsems[slot]).wait()

    def start_wb(i, slot):
        pltpu.async_copy(
            bufs[slot], out_hbm.at[pl.ds(base + i * _CHUNK, _CHUNK)],
            wsems[slot])

    def wait_wb(slot):
        pltpu.make_async_copy(
            bufs[slot], out_hbm.at[pl.ds(base, _CHUNK)], wsems[slot]).wait()

    start_gather(0, 0)
    start_gather(1, 1)
    wait_gather(0)
    start_wb(0, 0)

    def pair_body(p, carry):
        j = 1 + 2 * p
        wait_wb(0)
        start_gather(j + 1, 0)
        wait_gather(1)
        start_wb(j, 1)
        wait_wb(1)
        start_gather(j + 2, 1)
        wait_gather(0)
        start_wb(j + 1, 0)
        return carry

    lax.fori_loop(0, (_N_CHUNKS - 2) // 2, pair_body, 0)

    wait_gather(1)
    start_wb(_N_CHUNKS - 1, 1)
    wait_wb(0)
    wait_wb(1)


def kernel(token_ids, weight):
    flat = token_ids.reshape(-1)
    out = _gather_kernel(weight, flat)
    return out.reshape(token_ids.shape + (weight.shape[1],))
